# baseline (speedup 1.0000x reference)
import jax
import jax.numpy as jnp
from jax.experimental import pallas as pl
from jax.experimental.pallas import tpu as pltpu

_EPS = 1e-5
_W0 = 4
_PAD = _W0 + 1
_MB = 128


def _split_axes(t, n):
    vy = vx = 0
    for i in range(n):
        g = (t >> (2 * i)) & 3
        vy += (g >> 1) << (n - 1 - i)
        vx += (g & 1) << (n - 1 - i)
    return vy, vx


def _join_axes(vy, vx, n):
    t = 0
    for i in range(n):
        ry = (vy >> (n - 1 - i)) & 1
        rx = (vx >> (n - 1 - i)) & 1
        t += (2 * ry + rx) << (2 * i)
    return t


def _slab_src(t, n, dy, dx):
    vy, vx = _split_axes(t, n)
    lim = 1 << n

    def ax(v, d):
        v2 = v + d - 1
        if v2 < 0:
            return v2 + lim, -1
        if v2 >= lim:
            return v2 - lim, 1
        return v2, 0

    vy2, sy = ax(vy, dy)
    vx2, sx = ax(vx, dx)
    plane = _join_axes(vy2, vx2, n)
    kb = (sy + 1) * 3 + (sx + 1)
    off = _PAD + sy * _W0 + sx
    return plane, off, kb


def _masked(v, mv, kb):
    return v if kb == 4 else v * mv[:, kb:kb + 1]


def _bn_tanh(ys, g_ref, bt_ref):
    cnt = float(4 * ys[0].shape[0])
    s = ys[0].sum(axis=0, keepdims=True)
    for y in ys[1:]:
        s = s + y.sum(axis=0, keepdims=True)
    mean = s / cnt
    ds, sq = [], None
    for y in ys:
        d = y - mean
        ds.append(d)
        q = (d * d).sum(axis=0, keepdims=True)
        sq = q if sq is None else sq + q
    scale = g_ref[...] * jax.lax.rsqrt(sq / cnt + _EPS)
    bt = bt_ref[...]
    return [jnp.tanh(d * scale + bt) for d in ds]


def _pair_dots(lhs_of_k0, w_ref, Cin, C):
    ys = []
    for ry in range(2):
        for rx in range(2):
            p = 2 * ry + rx
            acc = None
            for a in range(2):
                k0 = (ry + a) * 3 + rx
                rhs = w_ref[k0 * Cin:(k0 + 2) * Cin, p * C:(p + 1) * C]
                d = jnp.dot(lhs_of_k0(k0), rhs,
                            preferred_element_type=jnp.float32)
                acc = d if acc is None else acc + d
            ys.append(acc)
    return ys


def _fused_kernel(x_ref, wp0_ref, g0_ref, bt0_ref, wp1_ref, g1_ref, bt1_ref,
                  wp2_ref, g2_ref, bt2_ref, wp3_ref, b3_ref, mask_ref,
                  o_ref, xp0, xb1, xb2, xb3, patch1, patch2, y2s, patch3):
    mv = mask_ref[...]
    z = jnp.zeros((_PAD, 512), jnp.float32)
    xp0[0:_PAD, :] = z
    xp0[_PAD + _MB:_PAD + _MB + _PAD, :] = z
    xb1[...] = jnp.zeros(xb1.shape, jnp.float32)
    xb2[...] = jnp.zeros(xb2.shape, jnp.float32)
    xb3[...] = jnp.zeros(xb3.shape, jnp.float32)

    for b in range(8):
        xb = x_ref[b * 512:(b + 1) * 512, :]
        xp0[_PAD + b * 16:_PAD + (b + 1) * 16, :] = jnp.transpose(xb)

    def l0_lhs(k0):
        sl = []
        for k in (k0, k0 + 1):
            dy, dx = divmod(k, 3)
            off = _PAD + (dy - 1) * _W0 + (dx - 1)
            sl.append(_masked(xp0[off:off + _MB, :], mv, k))
        return jnp.concatenate(sl, axis=1)

    ys = _bn_tanh(_pair_dots(l0_lhs, wp0_ref, 512, 256), g0_ref, bt0_ref)
    for p in range(4):
        xb1[p, _PAD:_PAD + _MB, :] = ys[p]

    for t in range(4):
        for k in range(9):
            dy, dx = divmod(k, 3)
            pl_, off, kb = _slab_src(t, 1, dy, dx)
            patch1[t * _MB:(t + 1) * _MB, k * 256:(k + 1) * 256] = \
                _masked(xb1[pl_, off:off + _MB, :], mv, kb)

    ys = _bn_tanh(_pair_dots(lambda k0: patch1[:, k0 * 256:(k0 + 2) * 256],
                             wp1_ref, 256, 128), g1_ref, bt1_ref)
    for p in range(4):
        for t in range(4):
            xb2[p * 4 + t, _PAD:_PAD + _MB, :] = \
                ys[p][t * _MB:(t + 1) * _MB, :]

    for c in range(2):
        for tc in range(8):
            t = c * 8 + tc
            for k in range(9):
                dy, dx = divmod(k, 3)
                pl_, off, kb = _slab_src(t, 2, dy, dx)
                patch2[tc * _MB:(tc + 1) * _MB, k * 128:(k + 1) * 128] = \
                    _masked(xb2[pl_, off:off + _MB, :], mv, kb)
        y2s[c * 1024:(c + 1) * 1024, :] = jnp.dot(
            patch2[...], wp2_ref[...], preferred_element_type=jnp.float32)

    ys = _bn_tanh([y2s[:, p * 64:(p + 1) * 64] for p in range(4)],
                  g2_ref, bt2_ref)
    for p in range(4):
        for t in range(16):
            xb3[p * 16 + t, _PAD:_PAD + _MB, :] = \
                ys[p][t * _MB:(t + 1) * _MB, :]

    b4 = jnp.concatenate([b3_ref[...]] * 4, axis=-1)
    for c in range(4):
        for tc in range(16):
            t = c * 16 + tc
            for k in range(9):
                dy, dx = divmod(k, 3)
                pl_, off, kb = _slab_src(t, 3, dy, dx)
                patch3[tc * _MB:(tc + 1) * _MB, k * 64:(k + 1) * 64] = \
                    _masked(xb3[pl_, off:off + _MB, :], mv, kb)
        y3 = jnp.dot(patch3[...], wp3_ref[...],
                     preferred_element_type=jnp.float32) + b4
        o_ref[c * 2048:(c + 1) * 2048, :] = \
            pl.reciprocal(1.0 + jnp.exp(-y3), approx=True)


def _whole(shape):
    return pl.BlockSpec(shape, lambda *_: (0,) * len(shape))


def kernel(x, wp0, b0, mask0, g0, bt0, wp1, b1, mask1, g1, bt1,
           wp2, b2, mask2, g2, bt2, wp3, b3, mask3):
    del b0, b1, b2, mask1, mask2, mask3
    xr = x.reshape(8 * 512, 16).astype(jnp.float32)
    args = (xr, wp0, g0, bt0, wp1, g1, bt1, wp2, g2, bt2, wp3, b3, mask0)
    out = pl.pallas_call(
        _fused_kernel,
        grid=(1,),
        in_specs=[_whole(a.shape) for a in args],
        out_specs=_whole((8192, 12)),
        out_shape=jax.ShapeDtypeStruct((8192, 12), jnp.float32),
        scratch_shapes=[
            pltpu.VMEM((_MB + 2 * _PAD, 512), jnp.float32),
            pltpu.VMEM((4, _MB + 2 * _PAD, 256), jnp.float32),
            pltpu.VMEM((16, _MB + 2 * _PAD, 128), jnp.float32),
            pltpu.VMEM((64, _MB + 2 * _PAD, 64), jnp.float32),
            pltpu.VMEM((512, 2304), jnp.float32),
            pltpu.VMEM((1024, 1152), jnp.float32),
            pltpu.VMEM((2048, 256), jnp.float32),
            pltpu.VMEM((2048, 576), jnp.float32),
        ],
        compiler_params=pltpu.CompilerParams(
            dimension_semantics=("arbitrary",)),
    )(*args)
    o = out.reshape(2, 2, 2, 2, 2, 2, 8, 4, 4, 2, 2, 3)
    o = jnp.transpose(o, (6, 11, 7, 4, 2, 0, 9, 8, 5, 3, 1, 10))
    return o.reshape(8, 3, 64, 64)


# speedup vs baseline: 1.5556x; 1.5556x over previous
import jax
import jax.numpy as jnp
from jax.experimental import pallas as pl
from jax.experimental.pallas import tpu as pltpu

_EPS = 1e-5
_W0 = 4
_PAD = _W0 + 1
_MB = 128
_W3 = 32
_PAD3 = 40


def _split_axes(t, n):
    vy = vx = 0
    for i in range(n):
        g = (t >> (2 * i)) & 3
        vy += (g >> 1) << (n - 1 - i)
        vx += (g & 1) << (n - 1 - i)
    return vy, vx


def _join_axes(vy, vx, n):
    t = 0
    for i in range(n):
        ry = (vy >> (n - 1 - i)) & 1
        rx = (vx >> (n - 1 - i)) & 1
        t += (2 * ry + rx) << (2 * i)
    return t


def _slab_src(t, n, dy, dx):
    vy, vx = _split_axes(t, n)
    lim = 1 << n

    def ax(v, d):
        v2 = v + d - 1
        if v2 < 0:
            return v2 + lim, -1
        if v2 >= lim:
            return v2 - lim, 1
        return v2, 0

    vy2, sy = ax(vy, dy)
    vx2, sx = ax(vx, dx)
    plane = _join_axes(vy2, vx2, n)
    kb = (sy + 1) * 3 + (sx + 1)
    off = _PAD + sy * _W0 + sx
    return plane, off, kb


def _masked(v, mv, kb):
    return v if kb == 4 else v * mv[:, kb:kb + 1]


def _bn_tanh(ys, g_ref, bt_ref):
    cnt = float(4 * ys[0].shape[0])
    s = ys[0].sum(axis=0, keepdims=True)
    for y in ys[1:]:
        s = s + y.sum(axis=0, keepdims=True)
    mean = s / cnt
    ds, sq = [], None
    for y in ys:
        d = y - mean
        ds.append(d)
        q = (d * d).sum(axis=0, keepdims=True)
        sq = q if sq is None else sq + q
    scale = g_ref[...] * jax.lax.rsqrt(sq / cnt + _EPS)
    bt = bt_ref[...]
    return [jnp.tanh(d * scale + bt) for d in ds]


def _pair_dots(lhs_of_k0, w_ref, Cin, C):
    ys = []
    for ry in range(2):
        for rx in range(2):
            p = 2 * ry + rx
            acc = None
            for a in range(2):
                k0 = (ry + a) * 3 + rx
                rhs = w_ref[k0 * Cin:(k0 + 2) * Cin, p * C:(p + 1) * C]
                d = jnp.dot(lhs_of_k0(k0), rhs,
                            preferred_element_type=jnp.float32)
                acc = d if acc is None else acc + d
            ys.append(acc)
    return ys


def _border_masks3(c):
    r = jax.lax.broadcasted_iota(jnp.int32, (2048, 1), 0) + c * 2048
    x3 = jax.lax.rem(r, _W3)
    y3 = jax.lax.rem(jax.lax.div(r, _W3), _W3)
    one = jnp.ones((2048, 1), jnp.float32)
    zero = jnp.zeros((2048, 1), jnp.float32)

    def cond(v, d):
        if d == 0:
            return v >= 1
        if d == 2:
            return v <= _W3 - 2
        return None

    cols = {}
    for dy in range(3):
        for dx in range(3):
            k = dy * 3 + dx
            if k == 4:
                cols[k] = None
                continue
            cy, cx = cond(y3, dy), cond(x3, dx)
            c2 = cy if cx is None else (cx if cy is None else cy & cx)
            cols[k] = jnp.where(c2, one, zero)
    return cols


def _fused_kernel(x_ref, wp0_ref, g0_ref, bt0_ref, wp1_ref, g1_ref, bt1_ref,
                  wp2_ref, g2_ref, bt2_ref, wp3_ref, b3_ref, mask_ref,
                  o_ref, xp0, xb1, xb2, xb3s, xp3, patch1, patch2, y2s,
                  patch3):
    mv = mask_ref[...]
    z = jnp.zeros((_PAD, 512), jnp.float32)
    xp0[0:_PAD, :] = z
    xp0[_PAD + _MB:_PAD + _MB + _PAD, :] = z
    xb1[...] = jnp.zeros(xb1.shape, jnp.float32)
    xb2[...] = jnp.zeros(xb2.shape, jnp.float32)
    z3 = jnp.zeros((_PAD3, 64), jnp.float32)
    xp3[0:_PAD3, :] = z3
    xp3[_PAD3 + 8192:_PAD3 + 8192 + _PAD3, :] = z3

    for b in range(8):
        xb = x_ref[b * 512:(b + 1) * 512, :]
        xp0[_PAD + b * 16:_PAD + (b + 1) * 16, :] = jnp.transpose(xb)

    def l0_lhs(k0):
        sl = []
        for k in (k0, k0 + 1):
            dy, dx = divmod(k, 3)
            off = _PAD + (dy - 1) * _W0 + (dx - 1)
            sl.append(_masked(xp0[off:off + _MB, :], mv, k))
        return jnp.concatenate(sl, axis=1)

    ys = _bn_tanh(_pair_dots(l0_lhs, wp0_ref, 512, 256), g0_ref, bt0_ref)
    for p in range(4):
        xb1[p, _PAD:_PAD + _MB, :] = ys[p]

    for t in range(4):
        for k in range(9):
            dy, dx = divmod(k, 3)
            pl_, off, kb = _slab_src(t, 1, dy, dx)
            patch1[t * _MB:(t + 1) * _MB, k * 256:(k + 1) * 256] = \
                _masked(xb1[pl_, off:off + _MB, :], mv, kb)

    ys = _bn_tanh(_pair_dots(lambda k0: patch1[:, k0 * 256:(k0 + 2) * 256],
                             wp1_ref, 256, 128), g1_ref, bt1_ref)
    for p in range(4):
        for t in range(4):
            xb2[p * 4 + t, _PAD:_PAD + _MB, :] = \
                ys[p][t * _MB:(t + 1) * _MB, :]

    for c in range(2):
        for tc in range(8):
            t = c * 8 + tc
            for k in range(9):
                dy, dx = divmod(k, 3)
                pl_, off, kb = _slab_src(t, 2, dy, dx)
                patch2[tc * _MB:(tc + 1) * _MB, k * 128:(k + 1) * 128] = \
                    _masked(xb2[pl_, off:off + _MB, :], mv, kb)
        y2s[c * 1024:(c + 1) * 1024, :] = jnp.dot(
            patch2[...], wp2_ref[...], preferred_element_type=jnp.float32)

    ys = _bn_tanh([y2s[:, p * 64:(p + 1) * 64] for p in range(4)],
                  g2_ref, bt2_ref)
    for p in range(4):
        for t in range(16):
            vy, vx = _split_axes(p * 16 + t, 3)
            xb3s[:, :, vy, :, vx, :] = \
                ys[p][t * _MB:(t + 1) * _MB, :].reshape(8, 4, 4, 64)
    xp3[_PAD3:_PAD3 + 8192, :] = xb3s[...].reshape(8192, 64)

    b4 = jnp.concatenate([b3_ref[...]] * 4, axis=-1)
    for c in range(4):
        bm = _border_masks3(c)
        for k in range(9):
            dy, dx = divmod(k, 3)
            off = _PAD3 + (dy - 1) * _W3 + (dx - 1) + c * 2048
            v = xp3[off:off + 2048, :]
            if bm[k] is not None:
                v = v * bm[k]
            patch3[:, k * 64:(k + 1) * 64] = v
        y3 = jnp.dot(patch3[...], wp3_ref[...],
                     preferred_element_type=jnp.float32) + b4
        o_ref[c * 2048:(c + 1) * 2048, :] = \
            pl.reciprocal(1.0 + jnp.exp(-y3), approx=True)


def _whole(shape):
    return pl.BlockSpec(shape, lambda *_: (0,) * len(shape))


def kernel(x, wp0, b0, mask0, g0, bt0, wp1, b1, mask1, g1, bt1,
           wp2, b2, mask2, g2, bt2, wp3, b3, mask3):
    del b0, b1, b2, mask1, mask2, mask3
    xr = x.reshape(8 * 512, 16).astype(jnp.float32)
    args = (xr, wp0, g0, bt0, wp1, g1, bt1, wp2, g2, bt2, wp3, b3, mask0)
    out = pl.pallas_call(
        _fused_kernel,
        grid=(1,),
        in_specs=[_whole(a.shape) for a in args],
        out_specs=_whole((8192, 12)),
        out_shape=jax.ShapeDtypeStruct((8192, 12), jnp.float32),
        scratch_shapes=[
            pltpu.VMEM((_MB + 2 * _PAD, 512), jnp.float32),
            pltpu.VMEM((4, _MB + 2 * _PAD, 256), jnp.float32),
            pltpu.VMEM((16, _MB + 2 * _PAD, 128), jnp.float32),
            pltpu.VMEM((8, 4, 8, 4, 8, 64), jnp.float32),
            pltpu.VMEM((8192 + 2 * _PAD3, 64), jnp.float32),
            pltpu.VMEM((512, 2304), jnp.float32),
            pltpu.VMEM((1024, 1152), jnp.float32),
            pltpu.VMEM((2048, 256), jnp.float32),
            pltpu.VMEM((2048, 576), jnp.float32),
        ],
        compiler_params=pltpu.CompilerParams(
            dimension_semantics=("arbitrary",),
            vmem_limit_bytes=64 * 1024 * 1024),
    )(*args)
    o = out.reshape(8, 32, 32, 2, 2, 3)
    o = jnp.transpose(o, (0, 5, 1, 3, 2, 4))
    return o.reshape(8, 3, 64, 64)


# speedup vs baseline: 1.6559x; 1.0645x over previous
import jax
import jax.numpy as jnp
from jax.experimental import pallas as pl
from jax.experimental.pallas import tpu as pltpu

_EPS = 1e-5
_W0 = 4
_PAD = _W0 + 1
_MB = 128
_W3 = 32
_PAD3 = 40


def _split_axes(t, n):
    vy = vx = 0
    for i in range(n):
        g = (t >> (2 * i)) & 3
        vy += (g >> 1) << (n - 1 - i)
        vx += (g & 1) << (n - 1 - i)
    return vy, vx


def _join_axes(vy, vx, n):
    t = 0
    for i in range(n):
        ry = (vy >> (n - 1 - i)) & 1
        rx = (vx >> (n - 1 - i)) & 1
        t += (2 * ry + rx) << (2 * i)
    return t


def _slab_src(t, n, dy, dx):
    vy, vx = _split_axes(t, n)
    lim = 1 << n

    def ax(v, d):
        v2 = v + d - 1
        if v2 < 0:
            return v2 + lim, -1
        if v2 >= lim:
            return v2 - lim, 1
        return v2, 0

    vy2, sy = ax(vy, dy)
    vx2, sx = ax(vx, dx)
    plane = _join_axes(vy2, vx2, n)
    kb = (sy + 1) * 3 + (sx + 1)
    off = _PAD + sy * _W0 + sx
    return plane, off, kb


def _masked(v, mv, kb):
    return v if kb == 4 else v * mv[kb]


def _bn_tanh(ys, g_ref, bt_ref):
    cnt = float(4 * ys[0].shape[0])
    s = ys[0].sum(axis=0, keepdims=True)
    for y in ys[1:]:
        s = s + y.sum(axis=0, keepdims=True)
    mean = s / cnt
    ds, sq = [], None
    for y in ys:
        d = y - mean
        ds.append(d)
        q = (d * d).sum(axis=0, keepdims=True)
        sq = q if sq is None else sq + q
    scale = g_ref[...] * jax.lax.rsqrt(sq / cnt + _EPS)
    bt = bt_ref[...]
    return [jnp.tanh(d * scale + bt) for d in ds]


def _pair_dots(lhs_of_k0, w_ref, Cin, C):
    ys = []
    for ry in range(2):
        for rx in range(2):
            p = 2 * ry + rx
            acc = None
            for a in range(2):
                k0 = (ry + a) * 3 + rx
                rhs = w_ref[k0 * Cin:(k0 + 2) * Cin, p * C:(p + 1) * C]
                d = jnp.dot(lhs_of_k0(k0), rhs,
                            preferred_element_type=jnp.float32)
                acc = d if acc is None else acc + d
            ys.append(acc)
    return ys


def _base_masks():
    r = jax.lax.broadcasted_iota(jnp.int32, (_MB, 1), 0)
    ix = r & 3
    iy = (r >> 2) & 3
    one = jnp.ones((_MB, 1), jnp.float32)
    zero = jnp.zeros((_MB, 1), jnp.float32)

    def cond(v, d):
        if d == 0:
            return v >= 1
        if d == 2:
            return v <= _W0 - 2
        return None

    cols = {}
    for dy in range(3):
        for dx in range(3):
            k = dy * 3 + dx
            if k == 4:
                continue
            cy, cx = cond(iy, dy), cond(ix, dx)
            c2 = cy if cx is None else (cx if cy is None else cy & cx)
            cols[k] = jnp.where(c2, one, zero)
    return cols


def _fused_kernel(x_ref, wp0_ref, g0_ref, bt0_ref, wp1_ref, g1_ref, bt1_ref,
                  wp2_ref, g2_ref, bt2_ref, wp3_ref, b3_ref,
                  o_ref, xp0, xb1, xb2, xb3s, xp3, patch1, patch2, y2s,
                  patch3):
    mv = _base_masks()
    z = jnp.zeros((_PAD, 512), jnp.float32)
    xp0[0:_PAD, :] = z
    xp0[_PAD + _MB:_PAD + _MB + _PAD, :] = z
    xb1[...] = jnp.zeros(xb1.shape, jnp.float32)
    xb2[...] = jnp.zeros(xb2.shape, jnp.float32)
    z3 = jnp.zeros((_PAD3, 64), jnp.float32)
    xp3[0:_PAD3, :] = z3
    xp3[_PAD3 + 8192:_PAD3 + 8192 + _PAD3, :] = z3

    for b in range(8):
        xb = x_ref[b * 512:(b + 1) * 512, :]
        xp0[_PAD + b * 16:_PAD + (b + 1) * 16, :] = jnp.transpose(xb)

    slabs0 = {}

    def l0_lhs(k0):
        sl = []
        for k in (k0, k0 + 1):
            if k not in slabs0:
                dy, dx = divmod(k, 3)
                off = _PAD + (dy - 1) * _W0 + (dx - 1)
                slabs0[k] = _masked(xp0[off:off + _MB, :], mv, k)
            sl.append(slabs0[k])
        return jnp.concatenate(sl, axis=1)

    ys = _bn_tanh(_pair_dots(l0_lhs, wp0_ref, 512, 256), g0_ref, bt0_ref)
    for p in range(4):
        xb1[p, _PAD:_PAD + _MB, :] = ys[p]

    for t in range(4):
        for k in range(9):
            dy, dx = divmod(k, 3)
            pl_, off, kb = _slab_src(t, 1, dy, dx)
            patch1[t * _MB:(t + 1) * _MB, k * 256:(k + 1) * 256] = \
                _masked(xb1[pl_, off:off + _MB, :], mv, kb)

    ys = _bn_tanh(_pair_dots(lambda k0: patch1[:, k0 * 256:(k0 + 2) * 256],
                             wp1_ref, 256, 128), g1_ref, bt1_ref)
    for p in range(4):
        for t in range(4):
            xb2[p * 4 + t, _PAD:_PAD + _MB, :] = \
                ys[p][t * _MB:(t + 1) * _MB, :]

    for c in range(2):
        for tc in range(8):
            t = c * 8 + tc
            for k in range(9):
                dy, dx = divmod(k, 3)
                pl_, off, kb = _slab_src(t, 2, dy, dx)
                patch2[tc * _MB:(tc + 1) * _MB, k * 128:(k + 1) * 128] = \
                    _masked(xb2[pl_, off:off + _MB, :], mv, kb)
        y2s[c * 1024:(c + 1) * 1024, :] = jnp.dot(
            patch2[...], wp2_ref[...], preferred_element_type=jnp.float32)

    ys = _bn_tanh([y2s[:, p * 64:(p + 1) * 64] for p in range(4)],
                  g2_ref, bt2_ref)
    for p in range(4):
        for t in range(16):
            vy, vx = _split_axes(p * 16 + t, 3)
            xb3s[:, :, vy, :, vx, :] = \
                ys[p][t * _MB:(t + 1) * _MB, :].reshape(8, 4, 4, 64)
    xp3[_PAD3:_PAD3 + 8192, :] = xb3s[...].reshape(8192, 64)

    b4 = jnp.concatenate([b3_ref[...]] * 4, axis=-1)
    for c in range(4):
        for k in range(9):
            dy, dx = divmod(k, 3)
            off = _PAD3 + (dy - 1) * _W3 + (dx - 1) + c * 2048
            patch3[:, :, :, k * 64:(k + 1) * 64] = \
                xp3[off:off + 2048, :].reshape(2, _W3, _W3, 64)
        patch3[:, 0, :, 0:192] = jnp.zeros((2, _W3, 192), jnp.float32)
        patch3[:, _W3 - 1, :, 384:576] = jnp.zeros((2, _W3, 192), jnp.float32)
        zc = jnp.zeros((2, _W3, 64), jnp.float32)
        for k in (0, 3, 6):
            patch3[:, :, 0, k * 64:(k + 1) * 64] = zc
        for k in (2, 5, 8):
            patch3[:, :, _W3 - 1, k * 64:(k + 1) * 64] = zc
        y3 = jnp.dot(patch3[...].reshape(2048, 576), wp3_ref[...],
                     preferred_element_type=jnp.float32) + b4
        o_ref[c * 2048:(c + 1) * 2048, :] = \
            pl.reciprocal(1.0 + jnp.exp(-y3), approx=True)


def _whole(shape):
    return pl.BlockSpec(shape, lambda *_: (0,) * len(shape))


def kernel(x, wp0, b0, mask0, g0, bt0, wp1, b1, mask1, g1, bt1,
           wp2, b2, mask2, g2, bt2, wp3, b3, mask3):
    del b0, b1, b2, mask0, mask1, mask2, mask3
    xr = x.reshape(8 * 512, 16).astype(jnp.float32)
    args = (xr, wp0, g0, bt0, wp1, g1, bt1, wp2, g2, bt2, wp3, b3)
    out = pl.pallas_call(
        _fused_kernel,
        grid=(1,),
        in_specs=[_whole(a.shape) for a in args],
        out_specs=_whole((8192, 12)),
        out_shape=jax.ShapeDtypeStruct((8192, 12), jnp.float32),
        scratch_shapes=[
            pltpu.VMEM((_MB + 2 * _PAD, 512), jnp.float32),
            pltpu.VMEM((4, _MB + 2 * _PAD, 256), jnp.float32),
            pltpu.VMEM((16, _MB + 2 * _PAD, 128), jnp.float32),
            pltpu.VMEM((8, 4, 8, 4, 8, 64), jnp.float32),
            pltpu.VMEM((8192 + 2 * _PAD3, 64), jnp.float32),
            pltpu.VMEM((512, 2304), jnp.float32),
            pltpu.VMEM((1024, 1152), jnp.float32),
            pltpu.VMEM((2048, 256), jnp.float32),
            pltpu.VMEM((2, _W3, _W3, 576), jnp.float32),
        ],
        compiler_params=pltpu.CompilerParams(
            dimension_semantics=("arbitrary",),
            vmem_limit_bytes=64 * 1024 * 1024),
    )(*args)
    o = out.reshape(8, 32, 32, 2, 2, 3)
    o = jnp.transpose(o, (0, 5, 1, 3, 2, 4))
    return o.reshape(8, 3, 64, 64)


# speedup vs baseline: 1.7279x; 1.0434x over previous
import jax
import jax.numpy as jnp
from jax.experimental import pallas as pl
from jax.experimental.pallas import tpu as pltpu

_EPS = 1e-5
_W0 = 4
_PAD = _W0 + 1
_MB = 128
_W3 = 32
_PAD3 = 40


def _split_axes(t, n):
    vy = vx = 0
    for i in range(n):
        g = (t >> (2 * i)) & 3
        vy += (g >> 1) << (n - 1 - i)
        vx += (g & 1) << (n - 1 - i)
    return vy, vx


def _join_axes(vy, vx, n):
    t = 0
    for i in range(n):
        ry = (vy >> (n - 1 - i)) & 1
        rx = (vx >> (n - 1 - i)) & 1
        t += (2 * ry + rx) << (2 * i)
    return t


def _slab_src(t, n, dy, dx):
    vy, vx = _split_axes(t, n)
    lim = 1 << n

    def ax(v, d):
        v2 = v + d - 1
        if v2 < 0:
            return v2 + lim, -1
        if v2 >= lim:
            return v2 - lim, 1
        return v2, 0

    vy2, sy = ax(vy, dy)
    vx2, sx = ax(vx, dx)
    plane = _join_axes(vy2, vx2, n)
    kb = (sy + 1) * 3 + (sx + 1)
    off = _PAD + sy * _W0 + sx
    return plane, off, kb


def _masked(v, mv, kb):
    return v if kb == 4 else v * mv[kb]


def _bn_tanh(ys, g_ref, bt_ref):
    cnt = float(4 * ys[0].shape[0])
    s = ys[0].sum(axis=0, keepdims=True)
    for y in ys[1:]:
        s = s + y.sum(axis=0, keepdims=True)
    mean = s / cnt
    ds, sq = [], None
    for y in ys:
        d = y - mean
        ds.append(d)
        q = (d * d).sum(axis=0, keepdims=True)
        sq = q if sq is None else sq + q
    scale = g_ref[...] * jax.lax.rsqrt(sq / cnt + _EPS)
    bt = bt_ref[...]
    return [jnp.tanh(d * scale + bt) for d in ds]


def _pair_dots(lhs_of_k0, w_ref, Cin, C):
    ys = []
    for ry in range(2):
        for rx in range(2):
            p = 2 * ry + rx
            acc = None
            for a in range(2):
                k0 = (ry + a) * 3 + rx
                rhs = w_ref[k0 * Cin:(k0 + 2) * Cin, p * C:(p + 1) * C]
                d = jnp.dot(lhs_of_k0(k0), rhs,
                            preferred_element_type=jnp.float32)
                acc = d if acc is None else acc + d
            ys.append(acc)
    return ys


def _base_masks():
    r = jax.lax.broadcasted_iota(jnp.int32, (_MB, 1), 0)
    ix = r & 3
    iy = (r >> 2) & 3
    one = jnp.ones((_MB, 1), jnp.float32)
    zero = jnp.zeros((_MB, 1), jnp.float32)

    def cond(v, d):
        if d == 0:
            return v >= 1
        if d == 2:
            return v <= _W0 - 2
        return None

    cols = {}
    for dy in range(3):
        for dx in range(3):
            k = dy * 3 + dx
            if k == 4:
                continue
            cy, cx = cond(iy, dy), cond(ix, dx)
            c2 = cy if cx is None else (cx if cy is None else cy & cx)
            cols[k] = jnp.where(c2, one, zero)
    return cols


def _fused_kernel(x_ref, wp0_ref, g0_ref, bt0_ref, wp1_ref, g1_ref, bt1_ref,
                  wp2_ref, g2_ref, bt2_ref, wp3_ref, b3_ref,
                  o_ref, xp0, xb1, xb2, xb3s, xp3, patch1, patch2, y2s,
                  patch3):
    mv = _base_masks()
    z = jnp.zeros((_PAD, 512), jnp.float32)
    xp0[0:_PAD, :] = z
    xp0[_PAD + _MB:_PAD + _MB + _PAD, :] = z
    xb1[...] = jnp.zeros(xb1.shape, jnp.float32)
    xb2[...] = jnp.zeros(xb2.shape, jnp.float32)
    z3 = jnp.zeros((_PAD3, 64), jnp.float32)
    xp3[0:_PAD3, :] = z3
    xp3[_PAD3 + 8192:_PAD3 + 8192 + _PAD3, :] = z3

    for b in range(8):
        xb = x_ref[b * 512:(b + 1) * 512, :]
        xp0[_PAD + b * 16:_PAD + (b + 1) * 16, :] = jnp.transpose(xb)

    slabs0 = {}

    def l0_lhs(k0):
        sl = []
        for k in (k0, k0 + 1):
            if k not in slabs0:
                dy, dx = divmod(k, 3)
                off = _PAD + (dy - 1) * _W0 + (dx - 1)
                slabs0[k] = _masked(xp0[off:off + _MB, :], mv, k)
            sl.append(slabs0[k])
        return jnp.concatenate(sl, axis=1)

    ys = _bn_tanh(_pair_dots(l0_lhs, wp0_ref, 512, 256), g0_ref, bt0_ref)
    for p in range(4):
        xb1[p, _PAD:_PAD + _MB, :] = ys[p]

    for t in range(4):
        for k in range(9):
            dy, dx = divmod(k, 3)
            pl_, off, kb = _slab_src(t, 1, dy, dx)
            patch1[t * _MB:(t + 1) * _MB, k * 256:(k + 1) * 256] = \
                _masked(xb1[pl_, off:off + _MB, :], mv, kb)

    ys = _bn_tanh(_pair_dots(lambda k0: patch1[:, k0 * 256:(k0 + 2) * 256],
                             wp1_ref, 256, 128), g1_ref, bt1_ref)
    for p in range(4):
        for t in range(4):
            xb2[p * 4 + t, _PAD:_PAD + _MB, :] = \
                ys[p][t * _MB:(t + 1) * _MB, :]

    for c in range(2):
        for tc in range(8):
            t = c * 8 + tc
            for k in range(9):
                dy, dx = divmod(k, 3)
                pl_, off, kb = _slab_src(t, 2, dy, dx)
                patch2[tc * _MB:(tc + 1) * _MB, k * 128:(k + 1) * 128] = \
                    _masked(xb2[pl_, off:off + _MB, :], mv, kb)
        y2s[c * 1024:(c + 1) * 1024, :] = jnp.dot(
            patch2[...], wp2_ref[...], preferred_element_type=jnp.float32)

    ys = _bn_tanh([y2s[:, p * 64:(p + 1) * 64] for p in range(4)],
                  g2_ref, bt2_ref)
    for p in range(4):
        for t in range(16):
            vy, vx = _split_axes(p * 16 + t, 3)
            xb3s[:, :, vy, :, vx, :] = \
                ys[p][t * _MB:(t + 1) * _MB, :].reshape(8, 4, 4, 64)
    xp3[_PAD3:_PAD3 + 8192, :] = xb3s[...].reshape(8192, 64)

    b4 = jnp.concatenate([b3_ref[...]] * 4, axis=-1)
    for c in range(4):
        for k in range(9):
            dy, dx = divmod(k, 3)
            off = _PAD3 + (dy - 1) * _W3 + (dx - 1) + c * 2048
            patch3[:, :, :, k * 64:(k + 1) * 64] = \
                xp3[off:off + 2048, :].reshape(2, _W3, _W3, 64)
        patch3[:, 0, :, 0:192] = jnp.zeros((2, _W3, 192), jnp.float32)
        patch3[:, _W3 - 1, :, 384:576] = jnp.zeros((2, _W3, 192), jnp.float32)
        zc = jnp.zeros((2, _W3, 64), jnp.float32)
        for k in (0, 3, 6):
            patch3[:, :, 0, k * 64:(k + 1) * 64] = zc
        for k in (2, 5, 8):
            patch3[:, :, _W3 - 1, k * 64:(k + 1) * 64] = zc
        y3 = jnp.dot(patch3[...].reshape(2048, 576), wp3_ref[...],
                     preferred_element_type=jnp.float32) + b4
        s3 = pl.reciprocal(1.0 + jnp.exp(-y3), approx=True)
        o_ref[:, c * 2048:(c + 1) * 2048] = jnp.transpose(s3)


def _whole(shape):
    return pl.BlockSpec(shape, lambda *_: (0,) * len(shape))


def kernel(x, wp0, b0, mask0, g0, bt0, wp1, b1, mask1, g1, bt1,
           wp2, b2, mask2, g2, bt2, wp3, b3, mask3):
    del b0, b1, b2, mask0, mask1, mask2, mask3
    xr = x.reshape(8 * 512, 16).astype(jnp.float32)
    args = (xr, wp0, g0, bt0, wp1, g1, bt1, wp2, g2, bt2, wp3, b3)
    out = pl.pallas_call(
        _fused_kernel,
        grid=(1,),
        in_specs=[_whole(a.shape) for a in args],
        out_specs=_whole((12, 8192)),
        out_shape=jax.ShapeDtypeStruct((12, 8192), jnp.float32),
        scratch_shapes=[
            pltpu.VMEM((_MB + 2 * _PAD, 512), jnp.float32),
            pltpu.VMEM((4, _MB + 2 * _PAD, 256), jnp.float32),
            pltpu.VMEM((16, _MB + 2 * _PAD, 128), jnp.float32),
            pltpu.VMEM((8, 4, 8, 4, 8, 64), jnp.float32),
            pltpu.VMEM((8192 + 2 * _PAD3, 64), jnp.float32),
            pltpu.VMEM((512, 2304), jnp.float32),
            pltpu.VMEM((1024, 1152), jnp.float32),
            pltpu.VMEM((2048, 256), jnp.float32),
            pltpu.VMEM((2, _W3, _W3, 576), jnp.float32),
        ],
        compiler_params=pltpu.CompilerParams(
            dimension_semantics=("arbitrary",),
            vmem_limit_bytes=64 * 1024 * 1024),
    )(*args)
    o = out.reshape(2, 2, 3, 8, 32, 32)
    o = jnp.transpose(o, (3, 2, 4, 0, 5, 1))
    return o.reshape(8, 3, 64, 64)
